# Initial kernel scaffold; baseline (speedup 1.0000x reference)
import jax
import jax.numpy as jnp
from jax.experimental import pallas as pl


def kernel(x, edge_index, edge_weight, W_own0, W_nbr0, b0, W_own1, W_nbr1, b1):
    raise NotImplementedError("write your pallas kernel here")


# speedup vs baseline: 5.1534x; 5.1534x over previous
import functools

import jax
import jax.numpy as jnp
from jax import lax
from jax.experimental import pallas as pl
from jax.experimental.pallas import tpu as pltpu
from jax.experimental.pallas import tpu_sc as plsc

def _lane_bcast(vec, lane):
    idx = jnp.full((16, 1), lane, jnp.int32)
    dnums = lax.GatherDimensionNumbers(
        offset_dims=(), collapsed_slice_dims=(0,), start_index_map=(0,))
    return lax.gather(vec, idx, dnums, (1,),
                      mode=lax.GatherScatterMode.PROMISE_IN_BOUNDS)


_NC = 2
_NS = 16
_NW = _NC * _NS
_CHUNK = 128


def _make_spmm(n_nodes, d, n_chunks):
    mesh = plsc.VectorSubcoreMesh(core_axis_name="c", subcore_axis_name="s",
                                  num_cores=_NC, num_subcores=_NS)
    rows_per_tile = (n_nodes // _NS) // 8 * 8
    tail_base = rows_per_tile * _NS
    tail_rows = n_nodes - tail_base

    @functools.partial(
        pl.kernel,
        out_type=jax.ShapeDtypeStruct((_NC, n_nodes, d), jnp.float32),
        mesh=mesh,
        scratch_types=[
            pltpu.VMEM_SHARED((n_nodes, d), jnp.float32),
            pltpu.VMEM((n_chunks, _CHUNK), jnp.int32),
            pltpu.VMEM((n_chunks, _CHUNK), jnp.int32),
            pltpu.VMEM((n_chunks * _CHUNK,), jnp.float32),
            pltpu.VMEM((_CHUNK, d), jnp.float32),
            pltpu.SemaphoreType.DMA,
        ],
        compiler_params=pltpu.CompilerParams(use_tc_tiling_on_sc=False),
    )
    def spmm(h_hbm, src_hbm, dst_hbm, w_hbm, zeros_hbm, out_hbm,
             acc_sh, src_v, dst_v, w_v, rows_v, sem):
        c = lax.axis_index("c")
        s = lax.axis_index("s")
        wid = c * _NS + s

        pltpu.sync_copy(zeros_hbm.at[pl.ds(s * rows_per_tile, rows_per_tile)],
                        acc_sh.at[pl.ds(s * rows_per_tile, rows_per_tile)])
        if tail_rows:
            @pl.when(s == _NS - 1)
            def _():
                pltpu.sync_copy(zeros_hbm.at[pl.ds(tail_base, tail_rows)],
                                acc_sh.at[pl.ds(tail_base, tail_rows)])

        pltpu.sync_copy(src_hbm.at[wid], src_v)
        pltpu.sync_copy(dst_hbm.at[wid], dst_v)
        pltpu.sync_copy(w_hbm.at[wid], w_v)
        plsc.subcore_barrier()

        def chunk_body(j, carry):
            pltpu.async_copy(h_hbm.at[src_v.at[j]], rows_v, sem).wait()

            for g in range(_CHUNK // 16):
                wv = w_v[pl.ds(j * _CHUNK + g * 16, 16)]
                for e16 in range(16):
                    e = g * 16 + e16
                    wb = _lane_bcast(wv, e16)
                    for f in range(d // 16):
                        sl = pl.ds(f * 16, 16)
                        rows_v[e, sl] = rows_v[e, sl] * wb

            pltpu.sync_copy(rows_v, acc_sh.at[dst_v.at[j]], add=True)
            return carry

        lax.fori_loop(0, n_chunks, chunk_body, 0)

        plsc.subcore_barrier()
        pltpu.sync_copy(acc_sh.at[pl.ds(s * rows_per_tile, rows_per_tile)],
                        out_hbm.at[c, pl.ds(s * rows_per_tile, rows_per_tile)])
        if tail_rows:
            @pl.when(s == _NS - 1)
            def _():
                pltpu.sync_copy(acc_sh.at[pl.ds(tail_base, tail_rows)],
                                out_hbm.at[c, pl.ds(tail_base, tail_rows)])

    return spmm


def _dense_in(x, W_nbr, W_own, b, block_n=1000):
    n, d_in = x.shape
    d_out = W_nbr.shape[1]

    def body(x_ref, wn_ref, wo_ref, b_ref, h_ref, own_ref):
        xb = x_ref[...]
        h_ref[...] = jnp.dot(xb, wn_ref[...], preferred_element_type=jnp.float32)
        own_ref[...] = (
            jnp.dot(xb, wo_ref[...], preferred_element_type=jnp.float32)
            + b_ref[...]
        )

    return pl.pallas_call(
        body,
        grid=(n // block_n,),
        in_specs=[
            pl.BlockSpec((block_n, d_in), lambda i: (i, 0)),
            pl.BlockSpec((d_in, d_out), lambda i: (0, 0)),
            pl.BlockSpec((d_in, d_out), lambda i: (0, 0)),
            pl.BlockSpec((1, d_out), lambda i: (0, 0)),
        ],
        out_specs=[
            pl.BlockSpec((block_n, d_out), lambda i: (i, 0)),
            pl.BlockSpec((block_n, d_out), lambda i: (i, 0)),
        ],
        out_shape=[
            jax.ShapeDtypeStruct((n, d_out), jnp.float32),
            jax.ShapeDtypeStruct((n, d_out), jnp.float32),
        ],
    )(x, W_nbr, W_own, b.reshape(1, d_out))


def _dense_mid(parts, own0, W_nbr, W_own, b, block_n=1000):
    _, n, d_in = parts.shape
    d_out = W_nbr.shape[1]

    def body(p_ref, own_ref, wn_ref, wo_ref, b_ref, h1_ref, own1_ref):
        h = jnp.tanh(p_ref[0] + p_ref[1] + own_ref[...])
        h1_ref[...] = jnp.dot(h, wn_ref[...], preferred_element_type=jnp.float32)
        own1_ref[...] = (
            jnp.dot(h, wo_ref[...], preferred_element_type=jnp.float32)
            + b_ref[...]
        )

    return pl.pallas_call(
        body,
        grid=(n // block_n,),
        in_specs=[
            pl.BlockSpec((2, block_n, d_in), lambda i: (0, i, 0)),
            pl.BlockSpec((block_n, d_in), lambda i: (i, 0)),
            pl.BlockSpec((d_in, d_out), lambda i: (0, 0)),
            pl.BlockSpec((d_in, d_out), lambda i: (0, 0)),
            pl.BlockSpec((1, d_out), lambda i: (0, 0)),
        ],
        out_specs=[
            pl.BlockSpec((block_n, d_out), lambda i: (i, 0)),
            pl.BlockSpec((block_n, d_out), lambda i: (i, 0)),
        ],
        out_shape=[
            jax.ShapeDtypeStruct((n, d_out), jnp.float32),
            jax.ShapeDtypeStruct((n, d_out), jnp.float32),
        ],
    )(parts, own0, W_nbr, W_own, b.reshape(1, d_out))


def _dense_out(parts, own, block_n=1000):
    _, n, d = parts.shape

    def body(p_ref, own_ref, o_ref):
        o_ref[...] = p_ref[0] + p_ref[1] + own_ref[...]

    return pl.pallas_call(
        body,
        grid=(n // block_n,),
        in_specs=[
            pl.BlockSpec((2, block_n, d), lambda i: (0, i, 0)),
            pl.BlockSpec((block_n, d), lambda i: (i, 0)),
        ],
        out_specs=pl.BlockSpec((block_n, d), lambda i: (i, 0)),
        out_shape=jax.ShapeDtypeStruct((n, d), jnp.float32),
    )(parts, own)


def kernel(x, edge_index, edge_weight, W_own0, W_nbr0, b0, W_own1, W_nbr1, b1):
    n, d_in = x.shape
    e = edge_weight.shape[0]
    d_hid = W_nbr0.shape[1]
    d_out = W_nbr1.shape[1]

    per_worker = -(-e // (_NW * _CHUNK)) * _CHUNK
    e_pad = per_worker * _NW
    n_chunks = per_worker // _CHUNK

    dst = edge_index[0].astype(jnp.int32)
    src = edge_index[1].astype(jnp.int32)
    w = edge_weight.astype(jnp.float32)
    pad = e_pad - e
    dst = jnp.concatenate([dst, jnp.zeros((pad,), jnp.int32)])
    src = jnp.concatenate([src, jnp.zeros((pad,), jnp.int32)])
    w = jnp.concatenate([w, jnp.zeros((pad,), jnp.float32)])
    dst = dst.reshape(_NW, n_chunks, _CHUNK)
    src = src.reshape(_NW, n_chunks, _CHUNK)
    w = w.reshape(_NW, n_chunks * _CHUNK)

    zeros_hid = jnp.zeros((n, d_hid), jnp.float32)
    zeros_out = jnp.zeros((n, d_out), jnp.float32)

    spmm0 = _make_spmm(n, d_hid, n_chunks)
    spmm1 = _make_spmm(n, d_out, n_chunks)

    h0, own0 = _dense_in(x, W_nbr0, W_own0, b0)
    parts0 = spmm0(h0, src, dst, w, zeros_hid)
    h1, own1 = _dense_mid(parts0, own0, W_nbr1, W_own1, b1)
    parts1 = spmm1(h1, src, dst, w, zeros_out)
    return _dense_out(parts1, own1)
